# baseline (speedup 1.0000x reference)
import jax
import jax.numpy as jnp
from jax.experimental import pallas as pl
from jax.experimental.pallas import tpu as pltpu

N_TOK = 32768
D_IN = 768
D_HID = 64
D_OUT = 768
TILE = 512
G = N_TOK // TILE
KI = 8
KO = 8


def _mlp_manual(x_hbm, w1t_ref, b1_ref, w2t_ref, b2_ref, out_hbm,
                xbuf, obuf, insem, outsem):
    def in_copy(i):
        slot = i % KI
        return pltpu.make_async_copy(
            x_hbm.at[pl.ds(i * TILE, TILE), :], xbuf.at[slot], insem.at[slot])

    def out_copy(i):
        slot = i % KO
        return pltpu.make_async_copy(
            obuf.at[slot], out_hbm.at[pl.ds(i * TILE, TILE), :],
            outsem.at[slot])

    w1 = w1t_ref[...].astype(jnp.bfloat16)
    w2 = w2t_ref[...].astype(jnp.bfloat16)
    b1v = b1_ref[...]
    b2v = b2_ref[...]

    for i in range(KI - 1):
        in_copy(i).start()

    for i in range(G):
        in_copy(i).wait()
        if i + KI - 1 < G:
            in_copy(i + KI - 1).start()
        if i >= KO:
            out_copy(i - KO).wait()
        xb = xbuf[i % KI].astype(jnp.bfloat16)
        h = jnp.maximum(
            jnp.dot(xb, w1, preferred_element_type=jnp.float32) + b1v, 0.0)
        obuf[i % KO] = jnp.dot(h.astype(jnp.bfloat16), w2,
                               preferred_element_type=jnp.float32) + b2v
        out_copy(i).start()

    for i in range(max(G - KO, 0), G):
        out_copy(i).wait()


def kernel(x, W1, b1, W2, b2):
    w1t = W1.T
    w2t = W2.T
    b1r = b1.reshape(1, D_HID)
    b2r = b2.reshape(1, D_OUT)

    out = pl.pallas_call(
        _mlp_manual,
        in_specs=[
            pl.BlockSpec(memory_space=pl.ANY),
            pl.BlockSpec((D_IN, D_HID), lambda: (0, 0)),
            pl.BlockSpec((1, D_HID), lambda: (0, 0)),
            pl.BlockSpec((D_HID, D_OUT), lambda: (0, 0)),
            pl.BlockSpec((1, D_OUT), lambda: (0, 0)),
        ],
        out_specs=pl.BlockSpec(memory_space=pl.ANY),
        out_shape=jax.ShapeDtypeStruct((N_TOK, D_OUT), jnp.float32),
        scratch_shapes=[
            pltpu.VMEM((KI, TILE, D_IN), jnp.float32),
            pltpu.VMEM((KO, TILE, D_OUT), jnp.float32),
            pltpu.SemaphoreType.DMA((KI,)),
            pltpu.SemaphoreType.DMA((KO,)),
        ],
        compiler_params=pltpu.CompilerParams(
            vmem_limit_bytes=128 * 1024 * 1024,
        ),
    )(x, w1t, b1r, w2t, b2r)

    aux = jnp.zeros((), dtype=jnp.float32)
    return (out, aux)


# speedup vs baseline: 1.0609x; 1.0609x over previous
import jax
import jax.numpy as jnp
from jax.experimental import pallas as pl
from jax.experimental.pallas import tpu as pltpu

N_TOK = 32768
D_IN = 768
D_HID = 64
D_OUT = 768
TILE = 1024
G = N_TOK // TILE
KI = 6
KO = 6


def _mlp_manual(x_hbm, w1t_ref, b1_ref, w2t_ref, b2_ref, out_hbm,
                xbuf, obuf, insem, outsem):
    def in_copy(i):
        slot = i % KI
        return pltpu.make_async_copy(
            x_hbm.at[pl.ds(i * TILE, TILE), :], xbuf.at[slot], insem.at[slot])

    def out_copy(i):
        slot = i % KO
        return pltpu.make_async_copy(
            obuf.at[slot], out_hbm.at[pl.ds(i * TILE, TILE), :],
            outsem.at[slot])

    w1 = w1t_ref[...].astype(jnp.bfloat16)
    w2 = w2t_ref[...].astype(jnp.bfloat16)
    b1v = b1_ref[...]
    b2v = b2_ref[...]

    for i in range(KI - 1):
        in_copy(i).start()

    for i in range(G):
        in_copy(i).wait()
        if i + KI - 1 < G:
            in_copy(i + KI - 1).start()
        if i >= KO:
            out_copy(i - KO).wait()
        xb = xbuf[i % KI].astype(jnp.bfloat16)
        h = jnp.maximum(
            jnp.dot(xb, w1, preferred_element_type=jnp.float32) + b1v, 0.0)
        obuf[i % KO] = jnp.dot(h.astype(jnp.bfloat16), w2,
                               preferred_element_type=jnp.float32) + b2v
        out_copy(i).start()

    for i in range(max(G - KO, 0), G):
        out_copy(i).wait()


def kernel(x, W1, b1, W2, b2):
    w1t = W1.T
    w2t = W2.T
    b1r = b1.reshape(1, D_HID)
    b2r = b2.reshape(1, D_OUT)

    out = pl.pallas_call(
        _mlp_manual,
        in_specs=[
            pl.BlockSpec(memory_space=pl.ANY),
            pl.BlockSpec((D_IN, D_HID), lambda: (0, 0)),
            pl.BlockSpec((1, D_HID), lambda: (0, 0)),
            pl.BlockSpec((D_HID, D_OUT), lambda: (0, 0)),
            pl.BlockSpec((1, D_OUT), lambda: (0, 0)),
        ],
        out_specs=pl.BlockSpec(memory_space=pl.ANY),
        out_shape=jax.ShapeDtypeStruct((N_TOK, D_OUT), jnp.float32),
        scratch_shapes=[
            pltpu.VMEM((KI, TILE, D_IN), jnp.float32),
            pltpu.VMEM((KO, TILE, D_OUT), jnp.float32),
            pltpu.SemaphoreType.DMA((KI,)),
            pltpu.SemaphoreType.DMA((KO,)),
        ],
        compiler_params=pltpu.CompilerParams(
            vmem_limit_bytes=128 * 1024 * 1024,
        ),
    )(x, w1t, b1r, w2t, b2r)

    aux = jnp.zeros((), dtype=jnp.float32)
    return (out, aux)
